# baseline (speedup 1.0000x reference)
import functools

import jax
import jax.numpy as jnp
from jax.experimental import pallas as pl
from jax.experimental.pallas import tpu as pltpu

_T_BLK = 1024


def _batched_experts_kernel(x_ref, r_ref, w0_hbm, b0_ref, w1_hbm, b1_ref,
                            o_ref, w0_buf, w1_buf, sem0, sem1):
    E = r_ref.shape[1]
    x = x_ref[...]
    r = r_ref[...]
    col = jax.lax.broadcasted_iota(jnp.int32, r.shape, 1)

    def copy_in(e, slot):
        c0 = pltpu.make_async_copy(w0_hbm.at[e], w0_buf.at[slot], sem0.at[slot])
        c1 = pltpu.make_async_copy(w1_hbm.at[e], w1_buf.at[slot], sem1.at[slot])
        c0.start()
        c1.start()
        return c0, c1

    pending = copy_in(0, 0)
    for e in range(E):
        if e + 1 < E:
            nxt = copy_in(e + 1, (e + 1) % 2)
        pending[0].wait()
        pending[1].wait()
        slot = e % 2
        h = jnp.dot(x, w0_buf[slot], preferred_element_type=jnp.float32)
        h += b0_ref[e]
        g = 0.5 * h * (1.0 + jax.lax.erf(h * 0.7071067811865476))
        yi = jnp.dot(g, w1_buf[slot], preferred_element_type=jnp.float32)
        yi += b1_ref[e]
        s = jnp.sum(jnp.where(col == e, r, 0.0), axis=1, keepdims=True)
        yi *= s
        if e == 0:
            o_ref[...] = yi
        else:
            o_ref[...] += yi
        if e + 1 < E:
            pending = nxt


@jax.jit
def kernel(x, routing_tensor, W0, b0, W1, b1):
    T, DIM = x.shape
    E = routing_tensor.shape[1]
    ED = W0.shape[2]

    grid = (T // _T_BLK,)
    out = pl.pallas_call(
        _batched_experts_kernel,
        grid=grid,
        in_specs=[
            pl.BlockSpec((_T_BLK, DIM), lambda t: (t, 0)),
            pl.BlockSpec((_T_BLK, E), lambda t: (t, 0)),
            pl.BlockSpec(memory_space=pl.ANY),
            pl.BlockSpec((E, 1, ED), lambda t: (0, 0, 0)),
            pl.BlockSpec(memory_space=pl.ANY),
            pl.BlockSpec((E, 1, DIM), lambda t: (0, 0, 0)),
        ],
        out_specs=pl.BlockSpec((_T_BLK, DIM), lambda t: (t, 0)),
        out_shape=jax.ShapeDtypeStruct((T, DIM), jnp.float32),
        scratch_shapes=[
            pltpu.VMEM((2, DIM, ED), jnp.float32),
            pltpu.VMEM((2, ED, DIM), jnp.float32),
            pltpu.SemaphoreType.DMA((2,)),
            pltpu.SemaphoreType.DMA((2,)),
        ],
        compiler_params=pltpu.CompilerParams(
            dimension_semantics=("arbitrary",),
            vmem_limit_bytes=62 * 1024 * 1024,
        ),
    )(x, routing_tensor, W0, b0, W1, b1)
    return out


# speedup vs baseline: 1.2466x; 1.2466x over previous
import functools

import jax
import jax.numpy as jnp
from jax.experimental import pallas as pl
from jax.experimental.pallas import tpu as pltpu

_T_BLK = 1024
_E_BLK = 1


def _batched_experts_kernel(x_ref, r_ref, w0_ref, b0_ref, w1_ref, b1_ref, o_ref):
    ep = pl.program_id(1)
    x = x_ref[...]
    r = r_ref[...]
    col = jax.lax.broadcasted_iota(jnp.int32, r.shape, 1)
    y = None
    for i in range(_E_BLK):
        h = jnp.dot(x, w0_ref[i], preferred_element_type=jnp.float32)
        h += b0_ref[i]
        g = 0.5 * h * (1.0 + jax.lax.erf(h * 0.7071067811865476))
        yi = jnp.dot(g, w1_ref[i], preferred_element_type=jnp.float32)
        yi += b1_ref[i]
        s = jnp.sum(jnp.where(col == ep * _E_BLK + i, r, 0.0),
                    axis=1, keepdims=True)
        yi *= s
        y = yi if y is None else y + yi

    @pl.when(ep == 0)
    def _init():
        o_ref[...] = y

    @pl.when(ep != 0)
    def _acc():
        o_ref[...] += y


@jax.jit
def kernel(x, routing_tensor, W0, b0, W1, b1):
    T, DIM = x.shape
    E = routing_tensor.shape[1]
    ED = W0.shape[2]

    grid = (T // _T_BLK, E // _E_BLK)
    out = pl.pallas_call(
        _batched_experts_kernel,
        grid=grid,
        in_specs=[
            pl.BlockSpec((_T_BLK, DIM), lambda t, e: (t, 0)),
            pl.BlockSpec((_T_BLK, E), lambda t, e: (t, 0)),
            pl.BlockSpec((_E_BLK, DIM, ED), lambda t, e: (e, 0, 0)),
            pl.BlockSpec((_E_BLK, 1, ED), lambda t, e: (e, 0, 0)),
            pl.BlockSpec((_E_BLK, ED, DIM), lambda t, e: (e, 0, 0)),
            pl.BlockSpec((_E_BLK, 1, DIM), lambda t, e: (e, 0, 0)),
        ],
        out_specs=pl.BlockSpec((_T_BLK, DIM), lambda t, e: (t, 0)),
        out_shape=jax.ShapeDtypeStruct((T, DIM), jnp.float32),
        compiler_params=pltpu.CompilerParams(
            dimension_semantics=("parallel", "arbitrary"),
            vmem_limit_bytes=62 * 1024 * 1024,

        ),
    )(x, routing_tensor, W0, b0, W1, b1)
    return out


# speedup vs baseline: 1.2470x; 1.0004x over previous
import jax
import jax.numpy as jnp
from jax.experimental import pallas as pl
from jax.experimental.pallas import tpu as pltpu

_T_BLK = 1024


def _batched_experts_kernel(x_ref, r_ref, w0_ref, b0_ref, w1_ref, b1_ref, o_ref):
    e = pl.program_id(1)
    h = jnp.dot(x_ref[...], w0_ref[0], preferred_element_type=jnp.float32)
    h += b0_ref[0]
    g = 0.5 * h * (1.0 + jax.lax.erf(h * 0.7071067811865476))
    y = jnp.dot(g, w1_ref[0], preferred_element_type=jnp.float32)
    y += b1_ref[0]
    r = r_ref[...]
    col = jax.lax.broadcasted_iota(jnp.int32, r.shape, 1)
    s = jnp.sum(jnp.where(col == e, r, 0.0), axis=1, keepdims=True)
    y *= s

    @pl.when(e == 0)
    def _init():
        o_ref[...] = y

    @pl.when(e != 0)
    def _acc():
        o_ref[...] += y


@jax.jit
def kernel(x, routing_tensor, W0, b0, W1, b1):
    T, DIM = x.shape
    E = routing_tensor.shape[1]
    ED = W0.shape[2]

    grid = (T // _T_BLK, E)
    out = pl.pallas_call(
        _batched_experts_kernel,
        grid=grid,
        in_specs=[
            pl.BlockSpec((_T_BLK, DIM), lambda t, e: (t, 0)),
            pl.BlockSpec((_T_BLK, E), lambda t, e: (t, 0)),
            pl.BlockSpec((1, DIM, ED), lambda t, e: (e, 0, 0)),
            pl.BlockSpec((1, 1, ED), lambda t, e: (e, 0, 0)),
            pl.BlockSpec((1, ED, DIM), lambda t, e: (e, 0, 0)),
            pl.BlockSpec((1, 1, DIM), lambda t, e: (e, 0, 0)),
        ],
        out_specs=pl.BlockSpec((_T_BLK, DIM), lambda t, e: (t, 0)),
        out_shape=jax.ShapeDtypeStruct((T, DIM), jnp.float32),
        compiler_params=pltpu.CompilerParams(
            dimension_semantics=("parallel", "arbitrary"),
            vmem_limit_bytes=62 * 1024 * 1024,
        ),
    )(x, routing_tensor, W0, b0, W1, b1)
    return out
